# Initial kernel scaffold; baseline (speedup 1.0000x reference)
import functools
import math

import jax
import jax.numpy as jnp
from jax import lax
from jax.experimental import pallas as pl
from jax.experimental.pallas import tpu as pltpu
from jax.experimental.pallas import tpu_sc as plsc

D_MODEL = 128
BATCH = 4096
SEQ = 50
TOTAL = BATCH * SEQ
COEFF = math.sqrt(float(D_MODEL))

NUM_CORES = 2
NUM_SUBCORES = 16
NW = NUM_CORES * NUM_SUBCORES
ROWS_PER_W = TOTAL // NW
CHUNK = 128
NCHUNK = ROWS_PER_W // CHUNK

_mesh = plsc.VectorSubcoreMesh(core_axis_name="c", subcore_axis_name="s")


@functools.partial(
    pl.kernel,
    mesh=_mesh,
    out_type=jax.ShapeDtypeStruct((NW, NCHUNK, CHUNK, D_MODEL), jnp.float32),
    scratch_types=[
        pltpu.VMEM((NCHUNK, CHUNK), jnp.int32),
        pltpu.VMEM((CHUNK, D_MODEL), jnp.float32),
        pltpu.VMEM((CHUNK, D_MODEL), jnp.float32),
        pltpu.SemaphoreType.DMA,
        pltpu.SemaphoreType.DMA,
        pltpu.SemaphoreType.DMA,
        pltpu.SemaphoreType.DMA,
    ],
)
def _emb_lookup(idx_hbm, table_hbm, out_hbm, idx_v, buf0, buf1,
                gs0, gs1, ss0, ss1):
    cid = lax.axis_index("c")
    sid = lax.axis_index("s")
    wid = sid * NUM_CORES + cid

    bufs = (buf0, buf1)
    gsems = (gs0, gs1)
    ssems = (ss0, ss1)

    pltpu.sync_copy(idx_hbm.at[wid], idx_v)

    def gather_start(g, b):
        pltpu.make_async_copy(
            table_hbm.at[idx_v.at[g]], bufs[b], gsems[b]).start()

    def gather_wait(b):
        pltpu.make_async_copy(
            table_hbm.at[idx_v.at[0]], bufs[b], gsems[b]).wait()

    def store_start(g, b):
        pltpu.make_async_copy(bufs[b], out_hbm.at[wid, g], ssems[b]).start()

    def store_wait(b):
        pltpu.make_async_copy(bufs[b], out_hbm.at[wid, 0], ssems[b]).wait()

    def scale(buf):
        def row(r, carry):
            for c in range(D_MODEL // 16):
                sl = pl.ds(c * 16, 16)
                buf[r, sl] = buf[r, sl] * COEFF
            return carry
        lax.fori_loop(0, CHUNK, row, 0)

    gather_start(0, 0)

    def outer(gg, carry):
        for b in range(2):
            g = gg * 2 + b
            nb = 1 - b

            @pl.when(g >= 1)
            def _():
                store_wait(nb)

            @pl.when(g + 1 < NCHUNK)
            def _():
                gather_start(g + 1, nb)

            gather_wait(b)
            scale(bufs[b])
            store_start(g, b)
        return carry

    lax.fori_loop(0, NCHUNK // 2, outer, 0)
    store_wait(0)
    store_wait(1)


def kernel(x, table):
    idx = x.reshape(NW, NCHUNK, CHUNK).astype(jnp.int32)
    out = _emb_lookup(idx, table.astype(jnp.float32))
    return out.reshape(BATCH, SEQ, D_MODEL)


# speedup vs baseline: 2.8795x; 2.8795x over previous
import functools
import math

import jax
import jax.numpy as jnp
from jax import lax
from jax.experimental import pallas as pl
from jax.experimental.pallas import tpu as pltpu
from jax.experimental.pallas import tpu_sc as plsc

D_MODEL = 128
BATCH = 4096
SEQ = 50
TOTAL = BATCH * SEQ
COEFF = math.sqrt(float(D_MODEL))

NUM_CORES = 2
NUM_SUBCORES = 16
NW = NUM_CORES * NUM_SUBCORES
ROWS_PER_W = TOTAL // NW
CHUNK = 128
NCHUNK = ROWS_PER_W // CHUNK

_mesh = plsc.VectorSubcoreMesh(core_axis_name="c", subcore_axis_name="s")


@functools.partial(
    pl.kernel,
    mesh=_mesh,
    out_type=jax.ShapeDtypeStruct((NW, NCHUNK, CHUNK, D_MODEL), jnp.float32),
    scratch_types=[
        pltpu.VMEM((NCHUNK, CHUNK), jnp.int32),
        pltpu.VMEM((CHUNK, D_MODEL), jnp.float32),
        pltpu.VMEM((CHUNK, D_MODEL), jnp.float32),
        pltpu.SemaphoreType.DMA,
        pltpu.SemaphoreType.DMA,
        pltpu.SemaphoreType.DMA,
        pltpu.SemaphoreType.DMA,
    ],
)
def _emb_lookup(idx_hbm, table_hbm, out_hbm, idx_v, buf0, buf1,
                gs0, gs1, ss0, ss1):
    cid = lax.axis_index("c")
    sid = lax.axis_index("s")
    wid = sid * NUM_CORES + cid

    bufs = (buf0, buf1)
    gsems = (gs0, gs1)
    ssems = (ss0, ss1)

    pltpu.sync_copy(idx_hbm.at[wid], idx_v)

    def gather_start(g, b):
        pltpu.make_async_copy(
            table_hbm.at[idx_v.at[g]], bufs[b], gsems[b]).start()

    def gather_wait(b):
        pltpu.make_async_copy(
            table_hbm.at[idx_v.at[0]], bufs[b], gsems[b]).wait()

    def store_start(g, b):
        pltpu.make_async_copy(bufs[b], out_hbm.at[wid, g], ssems[b]).start()

    def store_wait(b):
        pltpu.make_async_copy(bufs[b], out_hbm.at[wid, 0], ssems[b]).wait()

    def scale(buf):
        def row(r, carry):
            for c in range(D_MODEL // 16):
                sl = pl.ds(c * 16, 16)
                buf[r, sl] = buf[r, sl] * COEFF
            return carry
        lax.fori_loop(0, CHUNK, row, 0)

    gather_start(0, 0)

    def outer(gg, carry):
        for b in range(2):
            g = gg * 2 + b
            nb = 1 - b

            @pl.when(g >= 1)
            def _():
                store_wait(nb)

            @pl.when(g + 1 < NCHUNK)
            def _():
                gather_start(g + 1, nb)

            gather_wait(b)
            scale(bufs[b])
            store_start(g, b)
        return carry

    lax.fori_loop(0, NCHUNK // 2, outer, 0)
    store_wait(1)


def kernel(x, table):
    idx = x.reshape(NW, NCHUNK, CHUNK).astype(jnp.int32)
    out = _emb_lookup(idx, table.astype(jnp.float32))
    return out.reshape(BATCH, SEQ, D_MODEL)


# speedup vs baseline: 4.2796x; 1.4863x over previous
import functools
import math

import jax
import jax.numpy as jnp
from jax import lax
from jax.experimental import pallas as pl
from jax.experimental.pallas import tpu as pltpu
from jax.experimental.pallas import tpu_sc as plsc

D_MODEL = 128
BATCH = 4096
SEQ = 50
COEFF = math.sqrt(float(D_MODEL))

NUM_CORES = 2
NUM_SUBCORES = 16
NW = NUM_CORES * NUM_SUBCORES
NCHUNK = BATCH // NW

_mesh = plsc.VectorSubcoreMesh(core_axis_name="c", subcore_axis_name="s")


@functools.partial(
    pl.kernel,
    mesh=_mesh,
    out_type=jax.ShapeDtypeStruct((BATCH, SEQ, D_MODEL), jnp.float32),
    scratch_types=[
        pltpu.VMEM((NCHUNK, SEQ), jnp.int32),
        pltpu.VMEM((SEQ, D_MODEL), jnp.float32),
        pltpu.VMEM((SEQ, D_MODEL), jnp.float32),
        pltpu.SemaphoreType.DMA,
        pltpu.SemaphoreType.DMA,
        pltpu.SemaphoreType.DMA,
        pltpu.SemaphoreType.DMA,
    ],
)
def _emb_lookup(idx_hbm, table_hbm, out_hbm, idx_v, buf0, buf1,
                gs0, gs1, ss0, ss1):
    cid = lax.axis_index("c")
    sid = lax.axis_index("s")
    wid = sid * NUM_CORES + cid
    base = wid * NCHUNK

    bufs = (buf0, buf1)
    gsems = (gs0, gs1)
    ssems = (ss0, ss1)

    pltpu.sync_copy(idx_hbm.at[pl.ds(base, NCHUNK)], idx_v)

    def gather_start(g, b):
        pltpu.make_async_copy(
            table_hbm.at[idx_v.at[g]], bufs[b], gsems[b]).start()

    def gather_wait(b):
        pltpu.make_async_copy(
            table_hbm.at[idx_v.at[0]], bufs[b], gsems[b]).wait()

    def store_start(g, b):
        pltpu.make_async_copy(bufs[b], out_hbm.at[base + g], ssems[b]).start()

    def store_wait(b):
        pltpu.make_async_copy(bufs[b], out_hbm.at[base], ssems[b]).wait()

    def scale(buf):
        def row(r, carry):
            for c in range(D_MODEL // 16):
                sl = pl.ds(c * 16, 16)
                buf[r, sl] = buf[r, sl] * COEFF
            return carry
        lax.fori_loop(0, SEQ, row, 0)

    gather_start(0, 0)

    def outer(gg, carry):
        for b in range(2):
            g = gg * 2 + b
            nb = 1 - b

            @pl.when(g >= 1)
            def _():
                store_wait(nb)

            @pl.when(g + 1 < NCHUNK)
            def _():
                gather_start(g + 1, nb)

            gather_wait(b)
            scale(bufs[b])
            store_start(g, b)
        return carry

    lax.fori_loop(0, NCHUNK // 2, outer, 0)
    store_wait(1)


def kernel(x, table):
    return _emb_lookup(x.astype(jnp.int32), table.astype(jnp.float32))


# speedup vs baseline: 5.2332x; 1.2228x over previous
import functools
import math

import jax
import jax.numpy as jnp
from jax import lax
from jax.experimental import pallas as pl
from jax.experimental.pallas import tpu as pltpu
from jax.experimental.pallas import tpu_sc as plsc

D_MODEL = 128
BATCH = 4096
SEQ = 50
COEFF = math.sqrt(float(D_MODEL))

NUM_CORES = 2
NUM_SUBCORES = 16
NW = NUM_CORES * NUM_SUBCORES
BPW = BATCH // NW
EPC = 2
ROWS = EPC * SEQ
NCHUNK = BPW // EPC
NBUF = 4

_mesh = plsc.VectorSubcoreMesh(core_axis_name="c", subcore_axis_name="s")


@functools.partial(
    pl.kernel,
    mesh=_mesh,
    out_type=jax.ShapeDtypeStruct((BATCH, SEQ, D_MODEL), jnp.float32),
    scratch_types=[
        pltpu.VMEM((NCHUNK, ROWS), jnp.int32),
        pltpu.VMEM((ROWS, D_MODEL), jnp.float32),
        pltpu.VMEM((ROWS, D_MODEL), jnp.float32),
        pltpu.VMEM((ROWS, D_MODEL), jnp.float32),
        pltpu.VMEM((ROWS, D_MODEL), jnp.float32),
        pltpu.SemaphoreType.DMA,
        pltpu.SemaphoreType.DMA,
        pltpu.SemaphoreType.DMA,
        pltpu.SemaphoreType.DMA,
        pltpu.SemaphoreType.DMA,
        pltpu.SemaphoreType.DMA,
        pltpu.SemaphoreType.DMA,
        pltpu.SemaphoreType.DMA,
    ],
)
def _emb_lookup(idx_hbm, table_hbm, out_hbm, idx_v, buf0, buf1, buf2, buf3,
                gs0, gs1, gs2, gs3, ss0, ss1, ss2, ss3):
    cid = lax.axis_index("c")
    sid = lax.axis_index("s")
    wid = sid * NUM_CORES + cid
    base = wid * BPW

    bufs = (buf0, buf1, buf2, buf3)
    gsems = (gs0, gs1, gs2, gs3)
    ssems = (ss0, ss1, ss2, ss3)

    pltpu.sync_copy(idx_hbm.at[wid], idx_v)

    def gather_start(g, b):
        pltpu.make_async_copy(
            table_hbm.at[idx_v.at[g]], bufs[b], gsems[b]).start()

    def gather_wait(b):
        pltpu.make_async_copy(
            table_hbm.at[idx_v.at[0]], bufs[b], gsems[b]).wait()

    def store_start(g, b):
        for h in range(EPC):
            pltpu.make_async_copy(
                bufs[b].at[pl.ds(h * SEQ, SEQ)],
                out_hbm.at[base + g * EPC + h], ssems[b]).start()

    def store_wait(b):
        for h in range(EPC):
            pltpu.make_async_copy(
                bufs[b].at[pl.ds(h * SEQ, SEQ)],
                out_hbm.at[base], ssems[b]).wait()

    def scale(buf):
        @plsc.parallel_loop(0, ROWS, step=2, unroll=2)
        def _rows(r):
            for rr in range(2):
                for c in range(D_MODEL // 16):
                    sl = pl.ds(c * 16, 16)
                    buf[r + rr, sl] = buf[r + rr, sl] * COEFF

    for b in range(NBUF - 1):
        gather_start(b, b)

    def outer(gg, carry):
        for b in range(NBUF):
            g = gg * NBUF + b
            pb = (b + NBUF - 1) % NBUF

            @pl.when(g >= 1)
            def _():
                store_wait(pb)

            @pl.when(g + NBUF - 1 < NCHUNK)
            def _():
                gather_start(g + NBUF - 1, pb)

            gather_wait(b)
            scale(bufs[b])
            store_start(g, b)
        return carry

    lax.fori_loop(0, NCHUNK // NBUF, outer, 0)
    store_wait(NBUF - 1)


def kernel(x, table):
    idx = x.reshape(NW, NCHUNK, ROWS).astype(jnp.int32)
    return _emb_lookup(idx, table.astype(jnp.float32))
